# Initial kernel scaffold; baseline (speedup 1.0000x reference)
import jax
import jax.numpy as jnp
from jax.experimental import pallas as pl


def kernel(x, edge_index, batch, W1_rel, b1, W1_root, W2_rel, b2, W2_root, W3_rel, b3, W3_root, W4_rel, b4, W4_root, Wh, bh):
    raise NotImplementedError("write your pallas kernel here")


# speedup vs baseline: 4.6939x; 4.6939x over previous
import functools

import jax
import jax.numpy as jnp
from jax import lax
from jax.experimental import pallas as pl
from jax.experimental.pallas import tpu as pltpu
from jax.experimental.pallas import tpu_sc as plsc

N = 10000
E = 320000
G = 64
F = 128

NC = 2
NS = 16
CHUNK = 80
NP = 10240
ROWS_PER_TILE = NP // NS
ZROWS = 128

ROWB = 2000
NBLK = N // ROWB


def _segsum_kernel(ny, epw, split_edges):
    n_chunks = epw // CHUNK
    mesh = plsc.VectorSubcoreMesh(core_axis_name="c", subcore_axis_name="s",
                                  num_cores=NC, num_subcores=NS)

    @functools.partial(
        pl.kernel,
        out_type=jax.ShapeDtypeStruct((NC, NP, F), jnp.float32),
        mesh=mesh,
        scratch_types=[
            pltpu.VMEM((CHUNK,), jnp.int32),
            pltpu.VMEM((CHUNK,), jnp.int32),
            pltpu.VMEM((CHUNK, F), jnp.float32),
            pltpu.VMEM((ZROWS, F), jnp.float32),
            pltpu.VMEM_SHARED((NP, F), jnp.float32),
            pltpu.SemaphoreType.DMA,
        ],
    )
    def k(y_hbm, src_hbm, dst_hbm, out_hbm, src_v, dst_v, rows_v, zbuf, acc,
          sem):
        cid = lax.axis_index("c")
        sid = lax.axis_index("s")

        zv = jnp.zeros((16,), jnp.float32)

        def zero_row(i, carry):
            for j in range(F // 16):
                zbuf[i, pl.ds(j * 16, 16)] = zv
            return carry

        lax.fori_loop(0, ZROWS, zero_row, 0)
        row0 = sid * ROWS_PER_TILE
        for t in range(ROWS_PER_TILE // ZROWS):
            pltpu.sync_copy(zbuf, acc.at[pl.ds(row0 + t * ZROWS, ZROWS)])
        plsc.subcore_barrier()

        if split_edges:
            base = (sid * NC + cid) * epw
        else:
            base = sid * epw

        def body(g, carry):
            off = base + g * CHUNK
            pltpu.sync_copy(src_hbm.at[pl.ds(off, CHUNK)], src_v)
            if not split_edges:
                shift = jnp.broadcast_to(cid * N, (16,)).astype(jnp.int32)
                for j in range(CHUNK // 16):
                    src_v[pl.ds(j * 16, 16)] = src_v[pl.ds(j * 16, 16)] + shift
            pltpu.async_copy(y_hbm.at[src_v], rows_v, sem).wait()
            pltpu.sync_copy(dst_hbm.at[pl.ds(off, CHUNK)], dst_v)
            pltpu.sync_copy(rows_v, acc.at[dst_v], add=True)
            return carry

        lax.fori_loop(0, n_chunks, body, 0)

        plsc.subcore_barrier()
        pltpu.sync_copy(acc.at[pl.ds(row0, ROWS_PER_TILE)],
                        out_hbm.at[cid, pl.ds(row0, ROWS_PER_TILE)])

    return k


_seg_edge = _segsum_kernel(N, E // (NC * NS), True)
_seg_feat = _segsum_kernel(2 * N, E // NS, False)


def _dot(a, b):
    return jnp.dot(a, b, preferred_element_type=jnp.float32)


def _tc_a(x, W1_rel, W1_root, b1):
    def body(x_ref, wr_ref, wo_ref, b_ref, t_ref, r_ref):
        xb = x_ref[...]
        t_ref[...] = _dot(xb, wr_ref[...])
        r_ref[...] = _dot(xb, wo_ref[...]) + b_ref[...]

    return pl.pallas_call(
        body,
        grid=(NBLK,),
        in_specs=[
            pl.BlockSpec((ROWB, F), lambda i: (i, 0)),
            pl.BlockSpec((F, F), lambda i: (0, 0)),
            pl.BlockSpec((F, F), lambda i: (0, 0)),
            pl.BlockSpec((1, F), lambda i: (0, 0)),
        ],
        out_specs=[pl.BlockSpec((ROWB, F), lambda i: (i, 0))] * 2,
        out_shape=[jax.ShapeDtypeStruct((N, F), jnp.float32)] * 2,
    )(x, W1_rel, W1_root, b1.reshape(1, F))


def _tc_b(p1, r1):
    def body(p_ref, r_ref, o_ref):
        o_ref[...] = jnp.maximum(p_ref[0] + p_ref[1] + r_ref[...], 0.0)

    return pl.pallas_call(
        body,
        grid=(NBLK,),
        in_specs=[
            pl.BlockSpec((NC, ROWB, F), lambda i: (0, i, 0)),
            pl.BlockSpec((ROWB, F), lambda i: (i, 0)),
        ],
        out_specs=pl.BlockSpec((ROWB, F), lambda i: (i, 0)),
        out_shape=jax.ShapeDtypeStruct((N, F), jnp.float32),
    )(p1, r1)


def _tc_c(p2, h1, W2_rel, b2, W2_root):
    def body(p_ref, h_ref, wr_ref, b_ref, wo_ref, o_ref):
        a2 = p_ref[0] + p_ref[1]
        h2 = jnp.maximum(
            _dot(a2, wr_ref[...]) + b_ref[...] + _dot(h_ref[...], wo_ref[...]),
            0.0)
        o_ref[0] = h2[:, :F]
        o_ref[1] = h2[:, F:]

    return pl.pallas_call(
        body,
        grid=(NBLK,),
        in_specs=[
            pl.BlockSpec((NC, ROWB, F), lambda i: (0, i, 0)),
            pl.BlockSpec((ROWB, F), lambda i: (i, 0)),
            pl.BlockSpec((F, 2 * F), lambda i: (0, 0)),
            pl.BlockSpec((1, 2 * F), lambda i: (0, 0)),
            pl.BlockSpec((F, 2 * F), lambda i: (0, 0)),
        ],
        out_specs=pl.BlockSpec((2, ROWB, F), lambda i: (0, i, 0)),
        out_shape=jax.ShapeDtypeStruct((2, N, F), jnp.float32),
    )(p2, h1, W2_rel, b2.reshape(1, 2 * F), W2_root)


def _tc_d(q3, h2cat, W3_rel, b3, W3_root, W4_rel, W4_root, b4):
    def body(q_ref, h_ref, w3r_ref, b3_ref, w3o_ref, w4r_ref, w4o_ref,
             b4_ref, t_ref, r_ref):
        a3 = jnp.concatenate([q_ref[0], q_ref[1]], axis=1)
        h2 = jnp.concatenate([h_ref[0], h_ref[1]], axis=1)
        h3 = jnp.maximum(
            _dot(a3, w3r_ref[...]) + b3_ref[...] + _dot(h2, w3o_ref[...]),
            0.0)
        t_ref[...] = _dot(h3, w4r_ref[...])
        r_ref[...] = _dot(h3, w4o_ref[...]) + b4_ref[...]

    return pl.pallas_call(
        body,
        grid=(NBLK,),
        in_specs=[
            pl.BlockSpec((2, ROWB, F), lambda i: (0, i, 0)),
            pl.BlockSpec((2, ROWB, F), lambda i: (0, i, 0)),
            pl.BlockSpec((2 * F, 3 * F), lambda i: (0, 0)),
            pl.BlockSpec((1, 3 * F), lambda i: (0, 0)),
            pl.BlockSpec((2 * F, 3 * F), lambda i: (0, 0)),
            pl.BlockSpec((3 * F, F), lambda i: (0, 0)),
            pl.BlockSpec((3 * F, F), lambda i: (0, 0)),
            pl.BlockSpec((1, F), lambda i: (0, 0)),
        ],
        out_specs=[pl.BlockSpec((ROWB, F), lambda i: (i, 0))] * 2,
        out_shape=[jax.ShapeDtypeStruct((N, F), jnp.float32)] * 2,
    )(q3, h2cat, W3_rel, b3.reshape(1, 3 * F), W3_root, W4_rel, W4_root,
      b4.reshape(1, F))


def _tc_e(p4, r4, batch2d, Wh, bh):
    def body(p_ref, r_ref, b_ref, wh_ref, bh_ref, o_ref):
        h4 = p_ref[0] + p_ref[1] + r_ref[...]
        gid = lax.broadcasted_iota(jnp.int32, (G, N), 0)
        onehot = (gid == b_ref[...]).astype(jnp.float32)
        sums = _dot(onehot, h4)
        counts = jnp.sum(onehot, axis=1, keepdims=True)
        pooled = sums / jnp.maximum(counts, 1.0)
        o_ref[...] = _dot(pooled, wh_ref[...]) + bh_ref[...]

    return pl.pallas_call(
        body,
        grid=(1,),
        in_specs=[
            pl.BlockSpec((NC, N, F), lambda i: (0, 0, 0)),
            pl.BlockSpec((N, F), lambda i: (0, 0)),
            pl.BlockSpec((1, N), lambda i: (0, 0)),
            pl.BlockSpec((F, 1), lambda i: (0, 0)),
            pl.BlockSpec((1, 1), lambda i: (0, 0)),
        ],
        out_specs=pl.BlockSpec((G, 1), lambda i: (0, 0)),
        out_shape=jax.ShapeDtypeStruct((G, 1), jnp.float32),
    )(p4, r4, batch2d, Wh, bh.reshape(1, 1))


def kernel(x, edge_index, batch, W1_rel, b1, W1_root, W2_rel, b2, W2_root,
           W3_rel, b3, W3_root, W4_rel, b4, W4_root, Wh, bh):
    src = edge_index[0]
    dst = edge_index[1]

    t1, r1 = _tc_a(x, W1_rel, W1_root, b1)
    p1 = _seg_edge(t1, src, dst)
    h1 = _tc_b(p1, r1)

    p2 = _seg_edge(h1, src, dst)
    h2cat = _tc_c(p2, h1, W2_rel, b2, W2_root)

    q3 = _seg_feat(h2cat.reshape(2 * N, F), src, dst)
    t4, r4 = _tc_d(q3, h2cat, W3_rel, b3, W3_root, W4_rel, W4_root, b4)

    p4 = _seg_edge(t4, src, dst)
    return _tc_e(p4, r4, batch.reshape(1, N), Wh, bh)


# speedup vs baseline: 9.0313x; 1.9241x over previous
import functools

import jax
import jax.numpy as jnp
from jax import lax
from jax.experimental import pallas as pl
from jax.experimental.pallas import tpu as pltpu
from jax.experimental.pallas import tpu_sc as plsc

N = 10000
E = 320000
G = 64
F = 128

NC = 2
NS = 16
CHUNK = 80
NP = 10240
ROWS_PER_TILE = NP // NS
ZROWS = 128

ROWB = 2000
NBLK = N // ROWB


def _segsum_kernel(split_edges):
    epw = E // (NC * NS) if split_edges else E // NS
    n_chunks = epw // CHUNK
    n_pairs = n_chunks // 2
    mesh = plsc.VectorSubcoreMesh(core_axis_name="c", subcore_axis_name="s",
                                  num_cores=NC, num_subcores=NS)

    @functools.partial(
        pl.kernel,
        out_type=jax.ShapeDtypeStruct((NC, NP, F), jnp.float32),
        mesh=mesh,
        scratch_types=[
            pltpu.VMEM((CHUNK,), jnp.int32),
            pltpu.VMEM((CHUNK,), jnp.int32),
            pltpu.VMEM((CHUNK,), jnp.int32),
            pltpu.VMEM((CHUNK,), jnp.int32),
            pltpu.VMEM((CHUNK, F), jnp.float32),
            pltpu.VMEM((CHUNK, F), jnp.float32),
            pltpu.VMEM((ZROWS, F), jnp.float32),
            pltpu.VMEM_SHARED((NP, F), jnp.float32),
            pltpu.SemaphoreType.DMA,
            pltpu.SemaphoreType.DMA,
            pltpu.SemaphoreType.DMA,
            pltpu.SemaphoreType.DMA,
        ],
    )
    def k(y_hbm, src_hbm, dst_hbm, out_hbm, src0, dst0, src1, dst1,
          rows0, rows1, zbuf, acc, semg0, semg1, semi0, semi1):
        cid = lax.axis_index("c")
        sid = lax.axis_index("s")

        srcs = (src0, src1)
        dsts = (dst0, dst1)
        rows = (rows0, rows1)
        semg = (semg0, semg1)
        semi = (semi0, semi1)

        if split_edges:
            base = (sid * NC + cid) * epw
        else:
            base = sid * epw
        shift = jnp.broadcast_to(cid * N, (16,)).astype(jnp.int32)

        def idx_load(g, b):
            off = base + g * CHUNK
            pltpu.async_copy(src_hbm.at[pl.ds(off, CHUNK)], srcs[b], semi[b])
            pltpu.async_copy(dst_hbm.at[pl.ds(off, CHUNK)], dsts[b], semi[b])

        def idx_wait(g, b):
            off = base + g * CHUNK
            pltpu.make_async_copy(src_hbm.at[pl.ds(off, CHUNK)], srcs[b],
                                  semi[b]).wait()
            pltpu.make_async_copy(dst_hbm.at[pl.ds(off, CHUNK)], dsts[b],
                                  semi[b]).wait()
            if not split_edges:
                for j in range(CHUNK // 16):
                    srcs[b][pl.ds(j * 16, 16)] = (
                        srcs[b][pl.ds(j * 16, 16)] + shift)

        def gather(b):
            pltpu.async_copy(y_hbm.at[srcs[b]], rows[b], semg[b])

        def gather_wait(b):
            pltpu.make_async_copy(y_hbm.at[srcs[b]], rows[b], semg[b]).wait()

        def scatter(b):
            pltpu.sync_copy(rows[b], acc.at[dsts[b]], add=True)

        zv = jnp.zeros((16,), jnp.float32)

        def zero_row(i, carry):
            for j in range(F // 16):
                zbuf[i, pl.ds(j * 16, 16)] = zv
            return carry

        lax.fori_loop(0, ZROWS, zero_row, 0)
        row0 = sid * ROWS_PER_TILE
        for t in range(ROWS_PER_TILE // ZROWS):
            pltpu.sync_copy(zbuf, acc.at[pl.ds(row0 + t * ZROWS, ZROWS)])
        plsc.subcore_barrier()

        idx_load(0, 0)
        idx_wait(0, 0)
        gather(0)
        idx_load(1, 1)

        def pair(i, carry):
            g0 = 2 * i
            gather_wait(0)
            idx_wait(g0 + 1, 1)
            gather(1)
            scatter(0)

            @pl.when(g0 + 2 < n_chunks)
            def _():
                idx_load(g0 + 2, 0)

            gather_wait(1)

            @pl.when(g0 + 2 < n_chunks)
            def _():
                idx_wait(g0 + 2, 0)
                gather(0)

            scatter(1)

            @pl.when(g0 + 3 < n_chunks)
            def _():
                idx_load(g0 + 3, 1)

            return carry

        lax.fori_loop(0, n_pairs, pair, 0)

        if n_chunks % 2 == 1:
            gather_wait(0)
            scatter(0)

        plsc.subcore_barrier()
        pltpu.sync_copy(acc.at[pl.ds(row0, ROWS_PER_TILE)],
                        out_hbm.at[cid, pl.ds(row0, ROWS_PER_TILE)])

    return k


_seg_edge = _segsum_kernel(True)
_seg_feat = _segsum_kernel(False)


def _dot(a, b):
    return jnp.dot(a, b, preferred_element_type=jnp.float32)


def _tc_a(x, W1_rel, W1_root, b1):
    def body(x_ref, wr_ref, wo_ref, b_ref, t_ref, r_ref):
        xb = x_ref[...]
        t_ref[...] = _dot(xb, wr_ref[...])
        r_ref[...] = _dot(xb, wo_ref[...]) + b_ref[...]

    return pl.pallas_call(
        body,
        grid=(NBLK,),
        in_specs=[
            pl.BlockSpec((ROWB, F), lambda i: (i, 0)),
            pl.BlockSpec((F, F), lambda i: (0, 0)),
            pl.BlockSpec((F, F), lambda i: (0, 0)),
            pl.BlockSpec((1, F), lambda i: (0, 0)),
        ],
        out_specs=[pl.BlockSpec((ROWB, F), lambda i: (i, 0))] * 2,
        out_shape=[jax.ShapeDtypeStruct((N, F), jnp.float32)] * 2,
    )(x, W1_rel, W1_root, b1.reshape(1, F))


def _tc_b(p1, r1):
    def body(p_ref, r_ref, o_ref):
        o_ref[...] = jnp.maximum(p_ref[0] + p_ref[1] + r_ref[...], 0.0)

    return pl.pallas_call(
        body,
        grid=(NBLK,),
        in_specs=[
            pl.BlockSpec((NC, ROWB, F), lambda i: (0, i, 0)),
            pl.BlockSpec((ROWB, F), lambda i: (i, 0)),
        ],
        out_specs=pl.BlockSpec((ROWB, F), lambda i: (i, 0)),
        out_shape=jax.ShapeDtypeStruct((N, F), jnp.float32),
    )(p1, r1)


def _tc_c(p2, h1, W2_rel, b2, W2_root):
    def body(p_ref, h_ref, wr_ref, b_ref, wo_ref, o_ref):
        a2 = p_ref[0] + p_ref[1]
        h2 = jnp.maximum(
            _dot(a2, wr_ref[...]) + b_ref[...] + _dot(h_ref[...], wo_ref[...]),
            0.0)
        o_ref[0] = h2[:, :F]
        o_ref[1] = h2[:, F:]

    return pl.pallas_call(
        body,
        grid=(NBLK,),
        in_specs=[
            pl.BlockSpec((NC, ROWB, F), lambda i: (0, i, 0)),
            pl.BlockSpec((ROWB, F), lambda i: (i, 0)),
            pl.BlockSpec((F, 2 * F), lambda i: (0, 0)),
            pl.BlockSpec((1, 2 * F), lambda i: (0, 0)),
            pl.BlockSpec((F, 2 * F), lambda i: (0, 0)),
        ],
        out_specs=pl.BlockSpec((2, ROWB, F), lambda i: (0, i, 0)),
        out_shape=jax.ShapeDtypeStruct((2, N, F), jnp.float32),
    )(p2, h1, W2_rel, b2.reshape(1, 2 * F), W2_root)


def _tc_d(q3, h2cat, W3_rel, b3, W3_root, W4_rel, W4_root, b4):
    def body(q_ref, h_ref, w3r_ref, b3_ref, w3o_ref, w4r_ref, w4o_ref,
             b4_ref, t_ref, r_ref):
        a3 = jnp.concatenate([q_ref[0], q_ref[1]], axis=1)
        h2 = jnp.concatenate([h_ref[0], h_ref[1]], axis=1)
        h3 = jnp.maximum(
            _dot(a3, w3r_ref[...]) + b3_ref[...] + _dot(h2, w3o_ref[...]),
            0.0)
        t_ref[...] = _dot(h3, w4r_ref[...])
        r_ref[...] = _dot(h3, w4o_ref[...]) + b4_ref[...]

    return pl.pallas_call(
        body,
        grid=(NBLK,),
        in_specs=[
            pl.BlockSpec((2, ROWB, F), lambda i: (0, i, 0)),
            pl.BlockSpec((2, ROWB, F), lambda i: (0, i, 0)),
            pl.BlockSpec((2 * F, 3 * F), lambda i: (0, 0)),
            pl.BlockSpec((1, 3 * F), lambda i: (0, 0)),
            pl.BlockSpec((2 * F, 3 * F), lambda i: (0, 0)),
            pl.BlockSpec((3 * F, F), lambda i: (0, 0)),
            pl.BlockSpec((3 * F, F), lambda i: (0, 0)),
            pl.BlockSpec((1, F), lambda i: (0, 0)),
        ],
        out_specs=[pl.BlockSpec((ROWB, F), lambda i: (i, 0))] * 2,
        out_shape=[jax.ShapeDtypeStruct((N, F), jnp.float32)] * 2,
    )(q3, h2cat, W3_rel, b3.reshape(1, 3 * F), W3_root, W4_rel, W4_root,
      b4.reshape(1, F))


def _tc_e(p4, r4, batch2d, Wh, bh):
    def body(p_ref, r_ref, b_ref, wh_ref, bh_ref, o_ref):
        h4 = p_ref[0] + p_ref[1] + r_ref[...]
        gid = lax.broadcasted_iota(jnp.int32, (G, N), 0)
        onehot = (gid == b_ref[...]).astype(jnp.float32)
        sums = _dot(onehot, h4)
        counts = jnp.sum(onehot, axis=1, keepdims=True)
        pooled = sums / jnp.maximum(counts, 1.0)
        o_ref[...] = _dot(pooled, wh_ref[...]) + bh_ref[...]

    return pl.pallas_call(
        body,
        grid=(1,),
        in_specs=[
            pl.BlockSpec((NC, N, F), lambda i: (0, 0, 0)),
            pl.BlockSpec((N, F), lambda i: (0, 0)),
            pl.BlockSpec((1, N), lambda i: (0, 0)),
            pl.BlockSpec((F, 1), lambda i: (0, 0)),
            pl.BlockSpec((1, 1), lambda i: (0, 0)),
        ],
        out_specs=pl.BlockSpec((G, 1), lambda i: (0, 0)),
        out_shape=jax.ShapeDtypeStruct((G, 1), jnp.float32),
    )(p4, r4, batch2d, Wh, bh.reshape(1, 1))


def kernel(x, edge_index, batch, W1_rel, b1, W1_root, W2_rel, b2, W2_root,
           W3_rel, b3, W3_root, W4_rel, b4, W4_root, Wh, bh):
    src = edge_index[0]
    dst = edge_index[1]

    t1, r1 = _tc_a(x, W1_rel, W1_root, b1)
    p1 = _seg_edge(t1, src, dst)
    h1 = _tc_b(p1, r1)

    p2 = _seg_edge(h1, src, dst)
    h2cat = _tc_c(p2, h1, W2_rel, b2, W2_root)

    q3 = _seg_feat(h2cat.reshape(2 * N, F), src, dst)
    t4, r4 = _tc_d(q3, h2cat, W3_rel, b3, W3_root, W4_rel, W4_root, b4)

    p4 = _seg_edge(t4, src, dst)
    return _tc_e(p4, r4, batch.reshape(1, N), Wh, bh)
